# baseline (speedup 1.0000x reference)
import jax
import jax.numpy as jnp
from jax.experimental import pallas as pl
from jax.experimental.pallas import tpu as pltpu

N = 10000
D = 128
H = 128
E = 160000
ES = 32000
K = 10
C = 0.15


def _spmm(idx, vals, P, n):
    return jax.ops.segment_sum(vals[:, None] * P[idx[1]], idx[0], num_segments=n)


def _bn(x, g, b):
    m = x.mean(axis=0)
    v = x.var(axis=0)
    return g * (x - m) / jnp.sqrt(v + 1e-5) + b


def _loss_body(logits_ref, y_ref, out_ref):
    lg = logits_ref[...]
    y = y_ref[...]
    t = jnp.maximum(lg, 0.0) - lg * y + jnp.log1p(jnp.exp(-jnp.abs(lg)))
    out_ref[0, 0] = jnp.sum(t) / ES


def kernel(ap_idx, ap_vals, am_idx, am_vals, X, m0, edges, y,
           Ws0, Wx0, bn0_g, bn0_b, Ws1, Wx1, bn1_g, bn1_b, Wdec, bdec):
    Ws = [Ws0, Ws1]
    Wx = [Wx0, Wx1]
    gs = [bn0_g, bn1_g]
    bs = [bn0_b, bn1_b]
    prev = X
    cur = X
    for i in range(2):
        pred = cur @ Ws[i]
        P = pred
        M = m0
        tX = C * pred
        for _ in range(K):
            nP = _spmm(ap_idx, ap_vals, P, N) + _spmm(am_idx, am_vals, M, N) + tX
            nM = _spmm(am_idx, am_vals, P, N) + _spmm(ap_idx, ap_vals, M, N)
            P, M = nP, nM
        h = jnp.concatenate([P, M], axis=1) @ Wx[i]
        if i > 0:
            h = h + prev
        h = _bn(h, gs[i], bs[i])
        h = jnp.tanh(h)
        prev = h
        cur = h
    Z = cur
    a = (Z @ Wdec[:H, 0])[edges[0]]
    b = (Z @ Wdec[H:, 0])[edges[1]]
    logits = a + b + bdec[0]
    loss2d = pl.pallas_call(
        _loss_body,
        out_shape=jax.ShapeDtypeStruct((1, 1), jnp.float32),
        out_specs=pl.BlockSpec(memory_space=pltpu.SMEM),
    )(logits.reshape(ES // 128, 128), y.reshape(ES // 128, 128))
    return loss2d[0, 0]


# speedup vs baseline: 1.9199x; 1.9199x over previous
import functools

import jax
import jax.numpy as jnp
from jax import lax
from jax.experimental import pallas as pl
from jax.experimental.pallas import tpu as pltpu
from jax.experimental.pallas import tpu_sc as plsc

N = 10000
H = 128
E = 160000
ES = 32000
K = 10
C = 0.15

NC = 2
NS = 16
NW = NC * NS
NPT = E // NS
CH = 128
NCHUNK = -(-NPT // CH)
NPAD = NCHUNK * CH
NP = 10112
RPT = NP // NS
RBS = [(o, min(128, RPT - o)) for o in range(0, RPT, 128)]
EPT = ES // NW
ECH = 128
ENCH = 8
EPAD = ENCH * ECH

_mesh = plsc.VectorSubcoreMesh(core_axis_name="c", subcore_axis_name="s",
                               num_cores=NC, num_subcores=NS)


@functools.partial(
    pl.kernel,
    out_type=jax.ShapeDtypeStruct((2 * NP, H), jnp.float32),
    mesh=_mesh,
    scratch_types=[
        pltpu.VMEM((CH,), jnp.int32),
        pltpu.VMEM((CH,), jnp.int32),
        pltpu.VMEM((CH, 16), jnp.float32),
        pltpu.VMEM((CH, H), jnp.float32),
        pltpu.VMEM((CH, H), jnp.float32),
        pltpu.VMEM_SHARED((NP, H), jnp.float32),
    ],
)
def _diffuse_sc(S0, tX, colA, rowA, valxA, colB, rowB, valxB, Sout,
                colbuf, ribuf, vxbuf, rows, obuf, acc):
    c = lax.axis_index("c")
    s = lax.axis_index("s")
    cN = c * NP
    rbase = s * RPT

    def one_pass(col_h, row_h, valx_h, swap):
        def chunk(k, carry):
            pltpu.sync_copy(col_h.at[s, k], colbuf)
            pltpu.sync_copy(row_h.at[s, k], ribuf)
            pltpu.sync_copy(valx_h.at[s, k], vxbuf)

            def addc(g, cr):
                colbuf[pl.ds(g * 16, 16)] = colbuf[pl.ds(g * 16, 16)] + cN
                return cr
            lax.fori_loop(0, CH // 16, addc, 0)

            pltpu.sync_copy(Sout.at[colbuf], rows)

            def edges(g, cr):
                for u in range(4):
                    r = g * 4 + u
                    v = vxbuf[r, :]
                    src = [rows[r, pl.ds(q * 16, 16)] for q in range(8)]
                    for q in range(8):
                        dq = (q + 4) % 8 if swap else q
                        obuf[r, pl.ds(dq * 16, 16)] = src[q] * v
                return cr
            lax.fori_loop(0, CH // 4, edges, 0)

            pltpu.sync_copy(obuf, acc.at[ribuf], add=True)
            return carry
        lax.fori_loop(0, NCHUNK, chunk, 0)

    def refresh(src_h, first):
        for o, sz in RBS:
            off = rbase + o
            pltpu.sync_copy(src_h.at[pl.ds(cN + off, sz)] if first
                            else acc.at[pl.ds(off, sz)],
                            rows.at[pl.ds(0, sz)])
            pltpu.sync_copy(rows.at[pl.ds(0, sz)],
                            Sout.at[pl.ds(cN + off, sz)])
            pltpu.sync_copy(tX.at[pl.ds(cN + off, sz)],
                            rows.at[pl.ds(0, sz)])
            pltpu.sync_copy(rows.at[pl.ds(0, sz)],
                            acc.at[pl.ds(off, sz)])

    refresh(S0, True)
    plsc.subcore_barrier()

    def step(it, carry):
        one_pass(colA, rowA, valxA, False)
        one_pass(colB, rowB, valxB, True)
        plsc.subcore_barrier()
        refresh(S0, False)
        plsc.subcore_barrier()
        return carry
    lax.fori_loop(0, K, step, 0)


@functools.partial(
    pl.kernel,
    out_type=(jax.ShapeDtypeStruct((NW * EPAD, H), jnp.float32),
              jax.ShapeDtypeStruct((NW * EPAD, H), jnp.float32)),
    mesh=_mesh,
    scratch_types=[
        pltpu.VMEM((ECH,), jnp.int32),
        pltpu.VMEM((ECH, H), jnp.float32),
    ],
)
def _egather_sc(Z, e0, e1, f0, f1, ebuf, fbuf):
    c = lax.axis_index("c")
    s = lax.axis_index("s")
    w = c * NS + s
    for eh, fh in ((e0, f0), (e1, f1)):
        def chunk(k, carry):
            pltpu.sync_copy(eh.at[w, k], ebuf)
            pltpu.sync_copy(Z.at[ebuf], fbuf)
            pltpu.sync_copy(fbuf, fh.at[pl.dslice(w * EPAD + k * ECH, ECH)])
            return carry
        lax.fori_loop(0, ENCH, chunk, 0)


def _mm_body(x_ref, w_ref, o_ref):
    o_ref[...] = jnp.dot(x_ref[...], w_ref[...],
                         preferred_element_type=jnp.float32)


def _tc_matmul(x, w):
    n, dk = x.shape
    dout = w.shape[1]
    bn = 2000
    return pl.pallas_call(
        _mm_body,
        grid=(n // bn,),
        in_specs=[pl.BlockSpec((bn, dk), lambda i: (i, 0)),
                  pl.BlockSpec((dk, dout), lambda i: (0, 0))],
        out_specs=pl.BlockSpec((bn, dout), lambda i: (i, 0)),
        out_shape=jax.ShapeDtypeStruct((n, dout), jnp.float32),
    )(x, w)


def _post_body(add_prev, p_ref, m_ref, wx_ref, prev_ref, g_ref, b_ref, o_ref):
    t = jnp.dot(p_ref[...], wx_ref[...][:H], preferred_element_type=jnp.float32)
    t = t + jnp.dot(m_ref[...], wx_ref[...][H:], preferred_element_type=jnp.float32)
    if add_prev:
        t = t + prev_ref[...]
    mu = jnp.mean(t, axis=0)
    var = jnp.mean((t - mu) ** 2, axis=0)
    h = g_ref[...] * (t - mu) / jnp.sqrt(var + 1e-5) + b_ref[...]
    o_ref[...] = jnp.tanh(h)


def _tc_post(P, M, Wx, prev, g, b, add_prev):
    return pl.pallas_call(
        functools.partial(_post_body, add_prev),
        out_shape=jax.ShapeDtypeStruct((N, H), jnp.float32),
    )(P, M, Wx, prev, g.reshape(1, H), b.reshape(1, H))


def _loss_body(f0_ref, f1_ref, w_ref, y_ref, msk_ref, bd_ref, o_ref):
    i = pl.program_id(0)
    lg = jnp.dot(f0_ref[...], w_ref[...][:, 0],
                 preferred_element_type=jnp.float32)
    lg = lg + jnp.dot(f1_ref[...], w_ref[...][:, 1],
                      preferred_element_type=jnp.float32)
    lg = lg.reshape(y_ref.shape) + bd_ref[0]
    y = y_ref[...]
    t = jnp.maximum(lg, 0.0) - lg * y + jnp.log1p(jnp.exp(-jnp.abs(lg)))
    part = jnp.sum(t * msk_ref[...]) / ES

    @pl.when(i == 0)
    def _():
        o_ref[0, 0] = 0.0
    o_ref[0, 0] = o_ref[0, 0] + part


def _tc_loss(f0, f1, w2, ypad, mask, bdec):
    nb = 8
    rb = f0.shape[0] // nb
    yb = ypad.shape[0] // nb
    out = pl.pallas_call(
        _loss_body,
        grid=(nb,),
        out_shape=jax.ShapeDtypeStruct((1, 1), jnp.float32),
        in_specs=[pl.BlockSpec((rb, H), lambda i: (i, 0)),
                  pl.BlockSpec((rb, H), lambda i: (i, 0)),
                  pl.BlockSpec((H, 2), lambda i: (0, 0)),
                  pl.BlockSpec((yb, H), lambda i: (i, 0)),
                  pl.BlockSpec((yb, H), lambda i: (i, 0)),
                  pl.BlockSpec(memory_space=pltpu.SMEM)],
        out_specs=pl.BlockSpec((1, 1), lambda i: (0, 0),
                               memory_space=pltpu.SMEM),
    )(f0, f1, w2, ypad, mask, bdec)
    return out[0, 0]


def _prep_edges(idx, vals):
    cols = idx[1].reshape(NS, NPT)
    rows = idx[0].reshape(NS, NPT)
    v = vals.reshape(NS, NPT)
    pad = ((0, 0), (0, NPAD - NPT))
    cols = jnp.pad(cols, pad).reshape(NS, NCHUNK, CH)
    rows = jnp.pad(rows, pad).reshape(NS, NCHUNK, CH)
    v = jnp.pad(v, pad).reshape(NS, NCHUNK, CH)
    valx = jnp.broadcast_to(v[..., None], (NS, NCHUNK, CH, 16))
    return cols, rows, jnp.asarray(valx)


def _pack_state(pred, m0v):
    halves = []
    tx = []
    z = jnp.zeros((N, 64), jnp.float32)
    rpad = ((0, NP - N), (0, 0))
    for c in range(NC):
        sl = slice(64 * c, 64 * (c + 1))
        halves.append(jnp.pad(
            jnp.concatenate([pred[:, sl], m0v[:, sl]], axis=1), rpad))
        tx.append(jnp.pad(
            jnp.concatenate([C * pred[:, sl], z], axis=1), rpad))
    return (jnp.concatenate(halves, axis=0), jnp.concatenate(tx, axis=0))


def _unpack_state(S):
    P = jnp.concatenate([S[0:N, 0:64], S[NP:NP + N, 0:64]], axis=1)
    M = jnp.concatenate([S[0:N, 64:], S[NP:NP + N, 64:]], axis=1)
    return P, M


def kernel(ap_idx, ap_vals, am_idx, am_vals, X, m0, edges, y,
           Ws0, Wx0, bn0_g, bn0_b, Ws1, Wx1, bn1_g, bn1_b, Wdec, bdec):
    colA, rowA, valxA = _prep_edges(ap_idx, ap_vals)
    colB, rowB, valxB = _prep_edges(am_idx, am_vals)

    Ws = [Ws0, Ws1]
    Wx = [Wx0, Wx1]
    gs = [bn0_g, bn1_g]
    bs = [bn0_b, bn1_b]
    prev = X
    cur = X
    for i in range(2):
        pred = _tc_matmul(cur, Ws[i])
        S0, tX = _pack_state(pred, m0)
        S = _diffuse_sc(S0, tX, colA, rowA, valxA, colB, rowB, valxB)
        P, M = _unpack_state(S)
        h = _tc_post(P, M, Wx[i], prev, gs[i], bs[i], i > 0)
        prev = h
        cur = h
    Z = cur

    epad = ((0, 0), (0, EPAD - EPT))
    e0 = jnp.pad(edges[0].reshape(NW, EPT), epad).reshape(NW, ENCH, ECH)
    e1 = jnp.pad(edges[1].reshape(NW, EPT), epad).reshape(NW, ENCH, ECH)
    f0, f1 = _egather_sc(Z, e0, e1)

    ypad = jnp.pad(y.reshape(NW, EPT), epad).reshape(NW * EPAD // H, H)
    mask = jnp.pad(jnp.ones((NW, EPT), jnp.float32), epad)
    mask = mask.reshape(NW * EPAD // H, H)
    w2 = jnp.concatenate([Wdec[:H], Wdec[H:]], axis=1)
    return _tc_loss(f0, f1, w2, ypad, mask, bdec)


# speedup vs baseline: 2.2723x; 1.1836x over previous
import functools

import jax
import jax.numpy as jnp
from jax import lax
from jax.experimental import pallas as pl
from jax.experimental.pallas import tpu as pltpu
from jax.experimental.pallas import tpu_sc as plsc

N = 10000
H = 128
E = 160000
ES = 32000
K = 10
C = 0.15

NC = 2
NS = 16
NW = NC * NS
NPT = E // NS
CH = 64
NCHUNK = 158
NCHA = NCHUNK + 2
NP = 10112
RPT = NP // NS
RBS = [(o, min(CH, RPT - o)) for o in range(0, RPT, CH)]
EPT = ES // NW
ECH = 128
ENCH = 8
EPAD = ENCH * ECH

_mesh = plsc.VectorSubcoreMesh(core_axis_name="c", subcore_axis_name="s",
                               num_cores=NC, num_subcores=NS)


@functools.partial(
    pl.kernel,
    out_type=jax.ShapeDtypeStruct((2 * NP, H), jnp.float32),
    mesh=_mesh,
    scratch_types=[
        pltpu.VMEM((2, CH), jnp.int32),
        pltpu.VMEM((2, CH), jnp.int32),
        pltpu.VMEM((2, CH, 16), jnp.float32),
        pltpu.VMEM((2, CH, H), jnp.float32),
        pltpu.VMEM_SHARED((NP, H), jnp.float32),
        pltpu.SemaphoreType.DMA,
        pltpu.SemaphoreType.DMA,
        pltpu.SemaphoreType.DMA,
        pltpu.SemaphoreType.DMA,
    ],
)
def _diffuse_sc(S0, tX, colA, rowA, valxA, colB, rowB, valxB, Sout,
                colb, rib, vxb, rows, acc, isem0, isem1, gsem0, gsem1):
    c = lax.axis_index("c")
    s = lax.axis_index("s")
    rbase = s * RPT
    isems = (isem0, isem1)
    gsems = (gsem0, gsem1)

    def one_pass(col_h, row_h, valx_h, swap):
        def idx_issue(k, p):
            pltpu.async_copy(col_h.at[c, s, k], colb.at[p], isems[p])
            pltpu.async_copy(row_h.at[s, k], rib.at[p], isems[p])
            pltpu.async_copy(valx_h.at[s, k], vxb.at[p], isems[p])

        def idx_wait(k, p):
            pltpu.make_async_copy(col_h.at[c, s, k], colb.at[p], isems[p]).wait()
            pltpu.make_async_copy(row_h.at[s, k], rib.at[p], isems[p]).wait()
            pltpu.make_async_copy(valx_h.at[s, k], vxb.at[p], isems[p]).wait()

        def g_issue(p):
            pltpu.async_copy(Sout.at[colb.at[p]], rows.at[p], gsems[p])

        def g_wait(p):
            pltpu.make_async_copy(Sout.at[colb.at[p]], rows.at[p],
                                  gsems[p]).wait()

        def compute_scatter(p, swap):
            def edges(g, cr):
                for u in range(4):
                    r = g * 4 + u
                    v = vxb[p, r, :]
                    src = [rows[p, r, pl.ds(q * 16, 16)] for q in range(8)]
                    for q in range(8):
                        dq = (q + 4) % 8 if swap else q
                        rows[p, r, pl.ds(dq * 16, 16)] = src[q] * v
                return cr
            lax.fori_loop(0, CH // 4, edges, 0)
            pltpu.sync_copy(rows.at[p], acc.at[rib.at[p]], add=True)

        idx_issue(0, 0)
        idx_wait(0, 0)
        g_issue(0)
        idx_issue(1, 1)

        def pair(j, carry):
            k = 2 * j
            idx_wait(k + 1, 1)
            g_wait(0)
            g_issue(1)
            compute_scatter(0, swap)
            idx_issue(k + 2, 0)
            idx_wait(k + 2, 0)
            g_wait(1)
            g_issue(0)
            compute_scatter(1, swap)
            idx_issue(k + 3, 1)
            return carry
        lax.fori_loop(0, NCHUNK // 2, pair, 0)
        g_wait(0)
        idx_wait(NCHUNK + 1, 1)

    def refresh(src_h, first):
        for o, sz in RBS:
            off = rbase + o
            pltpu.sync_copy(src_h.at[pl.ds(cN_ + off, sz)] if first
                            else acc.at[pl.ds(off, sz)],
                            rows.at[0, pl.ds(0, sz)])
            pltpu.sync_copy(rows.at[0, pl.ds(0, sz)],
                            Sout.at[pl.ds(cN_ + off, sz)])
            pltpu.sync_copy(tX.at[pl.ds(cN_ + off, sz)],
                            rows.at[0, pl.ds(0, sz)])
            pltpu.sync_copy(rows.at[0, pl.ds(0, sz)],
                            acc.at[pl.ds(off, sz)])

    cN_ = c * NP
    refresh(S0, True)
    plsc.subcore_barrier()

    def step(it, carry):
        one_pass(colA, rowA, valxA, False)
        one_pass(colB, rowB, valxB, True)
        plsc.subcore_barrier()
        refresh(S0, False)
        plsc.subcore_barrier()
        return carry
    lax.fori_loop(0, K, step, 0)


@functools.partial(
    pl.kernel,
    out_type=(jax.ShapeDtypeStruct((NW * EPAD, H), jnp.float32),
              jax.ShapeDtypeStruct((NW * EPAD, H), jnp.float32)),
    mesh=_mesh,
    scratch_types=[
        pltpu.VMEM((ECH,), jnp.int32),
        pltpu.VMEM((ECH, H), jnp.float32),
    ],
)
def _egather_sc(Z, e0, e1, f0, f1, ebuf, fbuf):
    c = lax.axis_index("c")
    s = lax.axis_index("s")
    w = c * NS + s
    for eh, fh in ((e0, f0), (e1, f1)):
        def chunk(k, carry):
            pltpu.sync_copy(eh.at[w, k], ebuf)
            pltpu.sync_copy(Z.at[ebuf], fbuf)
            pltpu.sync_copy(fbuf, fh.at[pl.dslice(w * EPAD + k * ECH, ECH)])
            return carry
        lax.fori_loop(0, ENCH, chunk, 0)


def _mm_body(x_ref, w_ref, o_ref):
    o_ref[...] = jnp.dot(x_ref[...], w_ref[...],
                         preferred_element_type=jnp.float32)


def _tc_matmul(x, w):
    n, dk = x.shape
    dout = w.shape[1]
    bn = 2000
    return pl.pallas_call(
        _mm_body,
        grid=(n // bn,),
        in_specs=[pl.BlockSpec((bn, dk), lambda i: (i, 0)),
                  pl.BlockSpec((dk, dout), lambda i: (0, 0))],
        out_specs=pl.BlockSpec((bn, dout), lambda i: (i, 0)),
        out_shape=jax.ShapeDtypeStruct((n, dout), jnp.float32),
    )(x, w)


def _post_body(add_prev, p_ref, m_ref, wx_ref, prev_ref, g_ref, b_ref, o_ref):
    t = jnp.dot(p_ref[...], wx_ref[...][:H], preferred_element_type=jnp.float32)
    t = t + jnp.dot(m_ref[...], wx_ref[...][H:], preferred_element_type=jnp.float32)
    if add_prev:
        t = t + prev_ref[...]
    mu = jnp.mean(t, axis=0)
    var = jnp.mean((t - mu) ** 2, axis=0)
    h = g_ref[...] * (t - mu) / jnp.sqrt(var + 1e-5) + b_ref[...]
    o_ref[...] = jnp.tanh(h)


def _tc_post(P, M, Wx, prev, g, b, add_prev):
    return pl.pallas_call(
        functools.partial(_post_body, add_prev),
        out_shape=jax.ShapeDtypeStruct((N, H), jnp.float32),
    )(P, M, Wx, prev, g.reshape(1, H), b.reshape(1, H))


def _loss_body(f0_ref, f1_ref, w_ref, y_ref, msk_ref, bd_ref, o_ref):
    i = pl.program_id(0)
    lg = jnp.dot(f0_ref[...], w_ref[...][:, 0],
                 preferred_element_type=jnp.float32)
    lg = lg + jnp.dot(f1_ref[...], w_ref[...][:, 1],
                      preferred_element_type=jnp.float32)
    lg = lg.reshape(y_ref.shape) + bd_ref[0]
    y = y_ref[...]
    t = jnp.maximum(lg, 0.0) - lg * y + jnp.log1p(jnp.exp(-jnp.abs(lg)))
    part = jnp.sum(t * msk_ref[...]) / ES

    @pl.when(i == 0)
    def _():
        o_ref[0, 0] = 0.0
    o_ref[0, 0] = o_ref[0, 0] + part


def _tc_loss(f0, f1, w2, ypad, mask, bdec):
    nb = 8
    rb = f0.shape[0] // nb
    yb = ypad.shape[0] // nb
    out = pl.pallas_call(
        _loss_body,
        grid=(nb,),
        out_shape=jax.ShapeDtypeStruct((1, 1), jnp.float32),
        in_specs=[pl.BlockSpec((rb, H), lambda i: (i, 0)),
                  pl.BlockSpec((rb, H), lambda i: (i, 0)),
                  pl.BlockSpec((H, 2), lambda i: (0, 0)),
                  pl.BlockSpec((yb, H), lambda i: (i, 0)),
                  pl.BlockSpec((yb, H), lambda i: (i, 0)),
                  pl.BlockSpec(memory_space=pltpu.SMEM)],
        out_specs=pl.BlockSpec((1, 1), lambda i: (0, 0),
                               memory_space=pltpu.SMEM),
    )(f0, f1, w2, ypad, mask, bdec)
    return out[0, 0]


def _prep_edges(idx, vals):
    cols = idx[1].reshape(NS, NPT)
    rows = idx[0].reshape(NS, NPT)
    v = vals.reshape(NS, NPT)
    pad = ((0, 0), (0, NCHA * CH - NPT))
    cols = jnp.pad(cols, pad).reshape(NS, NCHA, CH)
    col2 = jnp.stack([cols, cols + NP], axis=0)
    rows = jnp.pad(rows, pad).reshape(NS, NCHA, CH)
    v = jnp.pad(v, pad).reshape(NS, NCHA, CH)
    valx = jnp.broadcast_to(v[..., None], (NS, NCHA, CH, 16))
    return col2, rows, jnp.asarray(valx)


def _pack_state(pred, m0v):
    halves = []
    tx = []
    z = jnp.zeros((N, 64), jnp.float32)
    rpad = ((0, NP - N), (0, 0))
    for c in range(NC):
        sl = slice(64 * c, 64 * (c + 1))
        halves.append(jnp.pad(
            jnp.concatenate([pred[:, sl], m0v[:, sl]], axis=1), rpad))
        tx.append(jnp.pad(
            jnp.concatenate([C * pred[:, sl], z], axis=1), rpad))
    return (jnp.concatenate(halves, axis=0), jnp.concatenate(tx, axis=0))


def _unpack_state(S):
    P = jnp.concatenate([S[0:N, 0:64], S[NP:NP + N, 0:64]], axis=1)
    M = jnp.concatenate([S[0:N, 64:], S[NP:NP + N, 64:]], axis=1)
    return P, M


def kernel(ap_idx, ap_vals, am_idx, am_vals, X, m0, edges, y,
           Ws0, Wx0, bn0_g, bn0_b, Ws1, Wx1, bn1_g, bn1_b, Wdec, bdec):
    colA, rowA, valxA = _prep_edges(ap_idx, ap_vals)
    colB, rowB, valxB = _prep_edges(am_idx, am_vals)

    Ws = [Ws0, Ws1]
    Wx = [Wx0, Wx1]
    gs = [bn0_g, bn1_g]
    bs = [bn0_b, bn1_b]
    prev = X
    cur = X
    for i in range(2):
        pred = _tc_matmul(cur, Ws[i])
        S0, tX = _pack_state(pred, m0)
        S = _diffuse_sc(S0, tX, colA, rowA, valxA, colB, rowB, valxB)
        P, M = _unpack_state(S)
        h = _tc_post(P, M, Wx[i], prev, gs[i], bs[i], i > 0)
        prev = h
        cur = h
    Z = cur

    epad = ((0, 0), (0, EPAD - EPT))
    e0 = jnp.pad(edges[0].reshape(NW, EPT), epad).reshape(NW, ENCH, ECH)
    e1 = jnp.pad(edges[1].reshape(NW, EPT), epad).reshape(NW, ENCH, ECH)
    f0, f1 = _egather_sc(Z, e0, e1)

    ypad = jnp.pad(y.reshape(NW, EPT), epad).reshape(NW * EPAD // H, H)
    mask = jnp.pad(jnp.ones((NW, EPT), jnp.float32), epad)
    mask = mask.reshape(NW * EPAD // H, H)
    w2 = jnp.concatenate([Wdec[:H], Wdec[H:]], axis=1)
    return _tc_loss(f0, f1, w2, ypad, mask, bdec)


# speedup vs baseline: 2.5760x; 1.1336x over previous
import functools

import jax
import jax.numpy as jnp
from jax import lax
from jax.experimental import pallas as pl
from jax.experimental.pallas import tpu as pltpu
from jax.experimental.pallas import tpu_sc as plsc

N = 10000
H = 128
E = 160000
ES = 32000
K = 10
C = 0.15

NC = 2
NS = 16
NW = NC * NS
NPT = E // NS
CH = 64
NCHUNK = 158
NCHA = NCHUNK + 2
NP = 10112
RPT = NP // NS
RBS = [(o, min(CH, RPT - o)) for o in range(0, RPT, CH)]
EPT = ES // NW
ECH = 128
ENCH = 8
EPAD = ENCH * ECH

_mesh = plsc.VectorSubcoreMesh(core_axis_name="c", subcore_axis_name="s",
                               num_cores=NC, num_subcores=NS)


@functools.partial(
    pl.kernel,
    out_type=jax.ShapeDtypeStruct((2 * NP, H), jnp.float32),
    mesh=_mesh,
    scratch_types=[
        pltpu.VMEM((2, CH), jnp.int32),
        pltpu.VMEM((2, CH), jnp.int32),
        pltpu.VMEM((2, CH), jnp.int32),
        pltpu.VMEM((2, CH, 16), jnp.float32),
        pltpu.VMEM((2, CH, H), jnp.float32),
        pltpu.VMEM((2, CH, H), jnp.float32),
        pltpu.VMEM_SHARED((NP, H), jnp.float32),
        pltpu.SemaphoreType.DMA,
        pltpu.SemaphoreType.DMA,
        pltpu.SemaphoreType.DMA,
        pltpu.SemaphoreType.DMA,
        pltpu.SemaphoreType.DMA,
        pltpu.SemaphoreType.DMA,
    ],
)
def _diffuse_sc(S0, tX, colA, rowA, valxA, colB, rowB, valxB, Sout,
                colb, rib, srib, vxb, rows, obuf, acc,
                isem0, isem1, gsem0, gsem1, ssem0, ssem1):
    c = lax.axis_index("c")
    s = lax.axis_index("s")
    rbase = s * RPT
    isems = (isem0, isem1)
    gsems = (gsem0, gsem1)
    ssems = (ssem0, ssem1)

    def one_pass(col_h, row_h, valx_h, swap):
        def idx_issue(k, p):
            pltpu.async_copy(col_h.at[c, s, k], colb.at[p], isems[p])
            pltpu.async_copy(row_h.at[s, k], rib.at[p], isems[p])
            pltpu.async_copy(valx_h.at[s, k], vxb.at[p], isems[p])

        def idx_wait(k, p):
            pltpu.make_async_copy(col_h.at[c, s, k], colb.at[p], isems[p]).wait()
            pltpu.make_async_copy(row_h.at[s, k], rib.at[p], isems[p]).wait()
            pltpu.make_async_copy(valx_h.at[s, k], vxb.at[p], isems[p]).wait()

        def g_issue(p):
            pltpu.async_copy(Sout.at[colb.at[p]], rows.at[p], gsems[p])

        def g_wait(p):
            pltpu.make_async_copy(Sout.at[colb.at[p]], rows.at[p],
                                  gsems[p]).wait()

        def compute(p, swap):
            def cpidx(g, cr):
                srib[p, pl.ds(g * 16, 16)] = rib[p, pl.ds(g * 16, 16)]
                return cr
            lax.fori_loop(0, CH // 16, cpidx, 0)

            def edges(g, cr):
                for u in range(4):
                    r = g * 4 + u
                    v = vxb[p, r, :]
                    src = [rows[p, r, pl.ds(q * 16, 16)] for q in range(8)]
                    for q in range(8):
                        dq = (q + 4) % 8 if swap else q
                        obuf[p, r, pl.ds(dq * 16, 16)] = src[q] * v
                return cr
            lax.fori_loop(0, CH // 4, edges, 0)

        def s_issue(p):
            pltpu.async_copy(obuf.at[p], acc.at[srib.at[p]], ssems[p],
                             add=True)

        def s_wait(p):
            pltpu.make_async_copy(obuf.at[p], acc.at[srib.at[p]],
                                  ssems[p]).wait()

        def head(k, p):
            idx_wait(k + 1, 1 - p)
            g_wait(p)
            g_issue(1 - p)

        def tail(k, p):
            s_issue(p)
            idx_issue(k + 2, p)

        idx_issue(0, 0)
        idx_wait(0, 0)
        g_issue(0)
        idx_issue(1, 1)
        head(0, 0)
        compute(0, swap)
        tail(0, 0)
        head(1, 1)
        compute(1, swap)
        tail(1, 1)

        def pair(j, carry):
            k = 2 * j
            head(k, 0)
            s_wait(0)
            compute(0, swap)
            tail(k, 0)
            head(k + 1, 1)
            s_wait(1)
            compute(1, swap)
            tail(k + 1, 1)
            return carry
        lax.fori_loop(1, NCHUNK // 2, pair, 0)
        g_wait(0)
        idx_wait(NCHUNK + 1, 1)
        s_wait(0)
        s_wait(1)

    def refresh(src_h, first):
        for o, sz in RBS:
            off = rbase + o
            pltpu.sync_copy(src_h.at[pl.ds(cN_ + off, sz)] if first
                            else acc.at[pl.ds(off, sz)],
                            rows.at[0, pl.ds(0, sz)])
            pltpu.sync_copy(rows.at[0, pl.ds(0, sz)],
                            Sout.at[pl.ds(cN_ + off, sz)])
            pltpu.sync_copy(tX.at[pl.ds(cN_ + off, sz)],
                            rows.at[0, pl.ds(0, sz)])
            pltpu.sync_copy(rows.at[0, pl.ds(0, sz)],
                            acc.at[pl.ds(off, sz)])

    cN_ = c * NP
    refresh(S0, True)
    plsc.subcore_barrier()

    def step(it, carry):
        one_pass(colA, rowA, valxA, False)
        one_pass(colB, rowB, valxB, True)
        plsc.subcore_barrier()
        refresh(S0, False)
        plsc.subcore_barrier()
        return carry
    lax.fori_loop(0, K, step, 0)


@functools.partial(
    pl.kernel,
    out_type=(jax.ShapeDtypeStruct((NW * EPAD, H), jnp.float32),
              jax.ShapeDtypeStruct((NW * EPAD, H), jnp.float32)),
    mesh=_mesh,
    scratch_types=[
        pltpu.VMEM((ECH,), jnp.int32),
        pltpu.VMEM((ECH, H), jnp.float32),
    ],
)
def _egather_sc(Z, e0, e1, f0, f1, ebuf, fbuf):
    c = lax.axis_index("c")
    s = lax.axis_index("s")
    w = c * NS + s
    for eh, fh in ((e0, f0), (e1, f1)):
        def chunk(k, carry):
            pltpu.sync_copy(eh.at[w, k], ebuf)
            pltpu.sync_copy(Z.at[ebuf], fbuf)
            pltpu.sync_copy(fbuf, fh.at[pl.dslice(w * EPAD + k * ECH, ECH)])
            return carry
        lax.fori_loop(0, ENCH, chunk, 0)


def _mm_body(x_ref, w_ref, o_ref):
    o_ref[...] = jnp.dot(x_ref[...], w_ref[...],
                         preferred_element_type=jnp.float32)


def _tc_matmul(x, w):
    n, dk = x.shape
    dout = w.shape[1]
    bn = 2000
    return pl.pallas_call(
        _mm_body,
        grid=(n // bn,),
        in_specs=[pl.BlockSpec((bn, dk), lambda i: (i, 0)),
                  pl.BlockSpec((dk, dout), lambda i: (0, 0))],
        out_specs=pl.BlockSpec((bn, dout), lambda i: (i, 0)),
        out_shape=jax.ShapeDtypeStruct((n, dout), jnp.float32),
    )(x, w)


def _post_body(add_prev, p_ref, m_ref, wx_ref, prev_ref, g_ref, b_ref, o_ref):
    t = jnp.dot(p_ref[...], wx_ref[...][:H], preferred_element_type=jnp.float32)
    t = t + jnp.dot(m_ref[...], wx_ref[...][H:], preferred_element_type=jnp.float32)
    if add_prev:
        t = t + prev_ref[...]
    mu = jnp.mean(t, axis=0)
    var = jnp.mean((t - mu) ** 2, axis=0)
    h = g_ref[...] * (t - mu) / jnp.sqrt(var + 1e-5) + b_ref[...]
    o_ref[...] = jnp.tanh(h)


def _tc_post(P, M, Wx, prev, g, b, add_prev):
    return pl.pallas_call(
        functools.partial(_post_body, add_prev),
        out_shape=jax.ShapeDtypeStruct((N, H), jnp.float32),
    )(P, M, Wx, prev, g.reshape(1, H), b.reshape(1, H))


def _loss_body(f0_ref, f1_ref, w_ref, y_ref, msk_ref, bd_ref, o_ref):
    i = pl.program_id(0)
    lg = jnp.dot(f0_ref[...], w_ref[...][:, 0],
                 preferred_element_type=jnp.float32)
    lg = lg + jnp.dot(f1_ref[...], w_ref[...][:, 1],
                      preferred_element_type=jnp.float32)
    lg = lg.reshape(y_ref.shape) + bd_ref[0]
    y = y_ref[...]
    t = jnp.maximum(lg, 0.0) - lg * y + jnp.log1p(jnp.exp(-jnp.abs(lg)))
    part = jnp.sum(t * msk_ref[...]) / ES

    @pl.when(i == 0)
    def _():
        o_ref[0, 0] = 0.0
    o_ref[0, 0] = o_ref[0, 0] + part


def _tc_loss(f0, f1, w2, ypad, mask, bdec):
    nb = 8
    rb = f0.shape[0] // nb
    yb = ypad.shape[0] // nb
    out = pl.pallas_call(
        _loss_body,
        grid=(nb,),
        out_shape=jax.ShapeDtypeStruct((1, 1), jnp.float32),
        in_specs=[pl.BlockSpec((rb, H), lambda i: (i, 0)),
                  pl.BlockSpec((rb, H), lambda i: (i, 0)),
                  pl.BlockSpec((H, 2), lambda i: (0, 0)),
                  pl.BlockSpec((yb, H), lambda i: (i, 0)),
                  pl.BlockSpec((yb, H), lambda i: (i, 0)),
                  pl.BlockSpec(memory_space=pltpu.SMEM)],
        out_specs=pl.BlockSpec((1, 1), lambda i: (0, 0),
                               memory_space=pltpu.SMEM),
    )(f0, f1, w2, ypad, mask, bdec)
    return out[0, 0]


def _prep_edges(idx, vals):
    cols = idx[1].reshape(NS, NPT)
    rows = idx[0].reshape(NS, NPT)
    v = vals.reshape(NS, NPT)
    pad = ((0, 0), (0, NCHA * CH - NPT))
    cols = jnp.pad(cols, pad).reshape(NS, NCHA, CH)
    col2 = jnp.stack([cols, cols + NP], axis=0)
    rows = jnp.pad(rows, pad).reshape(NS, NCHA, CH)
    v = jnp.pad(v, pad).reshape(NS, NCHA, CH)
    valx = jnp.broadcast_to(v[..., None], (NS, NCHA, CH, 16))
    return col2, rows, jnp.asarray(valx)


def _pack_state(pred, m0v):
    halves = []
    tx = []
    z = jnp.zeros((N, 64), jnp.float32)
    rpad = ((0, NP - N), (0, 0))
    for c in range(NC):
        sl = slice(64 * c, 64 * (c + 1))
        halves.append(jnp.pad(
            jnp.concatenate([pred[:, sl], m0v[:, sl]], axis=1), rpad))
        tx.append(jnp.pad(
            jnp.concatenate([C * pred[:, sl], z], axis=1), rpad))
    return (jnp.concatenate(halves, axis=0), jnp.concatenate(tx, axis=0))


def _unpack_state(S):
    P = jnp.concatenate([S[0:N, 0:64], S[NP:NP + N, 0:64]], axis=1)
    M = jnp.concatenate([S[0:N, 64:], S[NP:NP + N, 64:]], axis=1)
    return P, M


def kernel(ap_idx, ap_vals, am_idx, am_vals, X, m0, edges, y,
           Ws0, Wx0, bn0_g, bn0_b, Ws1, Wx1, bn1_g, bn1_b, Wdec, bdec):
    colA, rowA, valxA = _prep_edges(ap_idx, ap_vals)
    colB, rowB, valxB = _prep_edges(am_idx, am_vals)

    Ws = [Ws0, Ws1]
    Wx = [Wx0, Wx1]
    gs = [bn0_g, bn1_g]
    bs = [bn0_b, bn1_b]
    prev = X
    cur = X
    for i in range(2):
        pred = _tc_matmul(cur, Ws[i])
        S0, tX = _pack_state(pred, m0)
        S = _diffuse_sc(S0, tX, colA, rowA, valxA, colB, rowB, valxB)
        P, M = _unpack_state(S)
        h = _tc_post(P, M, Wx[i], prev, gs[i], bs[i], i > 0)
        prev = h
        cur = h
    Z = cur

    epad = ((0, 0), (0, EPAD - EPT))
    e0 = jnp.pad(edges[0].reshape(NW, EPT), epad).reshape(NW, ENCH, ECH)
    e1 = jnp.pad(edges[1].reshape(NW, EPT), epad).reshape(NW, ENCH, ECH)
    f0, f1 = _egather_sc(Z, e0, e1)

    ypad = jnp.pad(y.reshape(NW, EPT), epad).reshape(NW * EPAD // H, H)
    mask = jnp.pad(jnp.ones((NW, EPT), jnp.float32), epad)
    mask = mask.reshape(NW * EPAD // H, H)
    w2 = jnp.concatenate([Wdec[:H], Wdec[H:]], axis=1)
    return _tc_loss(f0, f1, w2, ypad, mask, bdec)


# speedup vs baseline: 2.6242x; 1.0187x over previous
import functools

import jax
import jax.numpy as jnp
from jax import lax
from jax.experimental import pallas as pl
from jax.experimental.pallas import tpu as pltpu
from jax.experimental.pallas import tpu_sc as plsc

N = 10000
H = 128
E = 160000
ES = 32000
K = 10
C = 0.15

NC = 2
NS = 16
NW = NC * NS
NPT = E // NS
CH = 64
NCHUNK = 158
NCHA = NCHUNK + 2
NP = 10112
RPT = NP // NS
RBS = [(o, min(CH, RPT - o)) for o in range(0, RPT, CH)]
EPT = ES // NW
ECH = 128
ENCH = 8
EPAD = ENCH * ECH

_mesh = plsc.VectorSubcoreMesh(core_axis_name="c", subcore_axis_name="s",
                               num_cores=NC, num_subcores=NS)


@functools.partial(
    pl.kernel,
    out_type=jax.ShapeDtypeStruct((2 * NP, H), jnp.float32),
    mesh=_mesh,
    scratch_types=[
        pltpu.VMEM((2, CH), jnp.int32),
        pltpu.VMEM((2, CH), jnp.int32),
        pltpu.VMEM((2, CH), jnp.int32),
        pltpu.VMEM((2, CH, 16), jnp.float32),
        pltpu.VMEM((2, CH, H), jnp.float32),
        pltpu.VMEM((2, CH, H), jnp.float32),
        pltpu.VMEM_SHARED((NP, H), jnp.float32),
        pltpu.SemaphoreType.DMA,
        pltpu.SemaphoreType.DMA,
        pltpu.SemaphoreType.DMA,
        pltpu.SemaphoreType.DMA,
        pltpu.SemaphoreType.DMA,
        pltpu.SemaphoreType.DMA,
        pltpu.SemaphoreType.DMA,
        pltpu.SemaphoreType.DMA,
    ],
)
def _diffuse_sc(S0, tX, colA, rowA, valxA, colB, rowB, valxB, Sout,
                colb, rib, srib, vxb, rows, obuf, acc,
                isem0, isem1, gsem0, gsem1, ssem0, ssem1, tsem0, tsem1):
    c = lax.axis_index("c")
    s = lax.axis_index("s")
    rbase = s * RPT
    isems = (isem0, isem1)
    gsems = (gsem0, gsem1)
    ssems = (ssem0, ssem1)
    tsems = (tsem0, tsem1)

    def one_pass(col_h, row_h, valx_h, swap):
        def idx_issue(k, p):
            pltpu.async_copy(col_h.at[c, s, k], colb.at[p], isems[p])
            pltpu.async_copy(row_h.at[s, k], rib.at[p], isems[p])
            pltpu.async_copy(valx_h.at[s, k], vxb.at[p], isems[p])

        def idx_wait(k, p):
            pltpu.make_async_copy(col_h.at[c, s, k], colb.at[p], isems[p]).wait()
            pltpu.make_async_copy(row_h.at[s, k], rib.at[p], isems[p]).wait()
            pltpu.make_async_copy(valx_h.at[s, k], vxb.at[p], isems[p]).wait()

        def g_issue(p):
            pltpu.async_copy(Sout.at[colb.at[p]], rows.at[p], gsems[p])

        def g_wait(p):
            pltpu.make_async_copy(Sout.at[colb.at[p]], rows.at[p],
                                  gsems[p]).wait()

        def compute(p, swap):
            @plsc.parallel_loop(0, CH // 16, 1, unroll=4)
            def cpidx(g):
                srib[p, pl.ds(g * 16, 16)] = rib[p, pl.ds(g * 16, 16)]

            @plsc.parallel_loop(0, CH, 1, unroll=4)
            def edges(r):
                v = vxb[p, r, :]
                src = [rows[p, r, pl.ds(q * 16, 16)] for q in range(8)]
                for q in range(8):
                    dq = (q + 4) % 8 if swap else q
                    obuf[p, r, pl.ds(dq * 16, 16)] = src[q] * v

        def s_issue(p):
            pltpu.async_copy(obuf.at[p], acc.at[srib.at[p]], ssems[p],
                             add=True)

        def s_wait(p):
            pltpu.make_async_copy(obuf.at[p], acc.at[srib.at[p]],
                                  ssems[p]).wait()

        def head(k, p):
            idx_wait(k + 1, 1 - p)
            g_wait(p)
            g_issue(1 - p)

        def tail(k, p):
            s_issue(p)
            idx_issue(k + 2, p)

        idx_issue(0, 0)
        idx_wait(0, 0)
        g_issue(0)
        idx_issue(1, 1)
        head(0, 0)
        compute(0, swap)
        tail(0, 0)
        head(1, 1)
        compute(1, swap)
        tail(1, 1)

        def pair(j, carry):
            k = 2 * j
            head(k, 0)
            s_wait(0)
            compute(0, swap)
            tail(k, 0)
            head(k + 1, 1)
            s_wait(1)
            compute(1, swap)
            tail(k + 1, 1)
            return carry
        lax.fori_loop(1, NCHUNK // 2, pair, 0)
        g_wait(0)
        idx_wait(NCHUNK + 1, 1)
        s_wait(0)
        s_wait(1)

    def refresh(src_h, first):
        def rd(i, o, sz, p):
            srcref = (src_h.at[pl.ds(cN_ + rbase + o, sz)] if first
                      else acc.at[pl.ds(rbase + o, sz)])
            return pltpu.make_async_copy(srcref, rows.at[p, pl.ds(0, sz)],
                                         isems[p])
        def wr(o, sz, p):
            return pltpu.make_async_copy(rows.at[p, pl.ds(0, sz)],
                                         Sout.at[pl.ds(cN_ + rbase + o, sz)],
                                         gsems[p])
        def tx(o, sz, p):
            return pltpu.make_async_copy(tX.at[pl.ds(cN_ + rbase + o, sz)],
                                         obuf.at[p, pl.ds(0, sz)], tsems[p])
        def ac(o, sz, p):
            return pltpu.make_async_copy(obuf.at[p, pl.ds(0, sz)],
                                         acc.at[pl.ds(rbase + o, sz)],
                                         ssems[p])

        nrb = len(RBS)
        for i, (o, sz) in enumerate(RBS):
            p = i % 2
            if i >= 2:
                po, psz = RBS[i - 2]
                wr(po, psz, p).wait()
                ac(po, psz, p).wait()
            rd(i, o, sz, p).start()
            tx(o, sz, p).start()
            rd(i, o, sz, p).wait()
            wr(o, sz, p).start()
            tx(o, sz, p).wait()
            ac(o, sz, p).start()
        for i in (nrb - 2, nrb - 1):
            o, sz = RBS[i]
            p = i % 2
            wr(o, sz, p).wait()
            ac(o, sz, p).wait()

    cN_ = c * NP
    refresh(S0, True)
    plsc.subcore_barrier()

    def step(it, carry):
        one_pass(colA, rowA, valxA, False)
        one_pass(colB, rowB, valxB, True)
        plsc.subcore_barrier()
        refresh(S0, False)
        plsc.subcore_barrier()
        return carry
    lax.fori_loop(0, K, step, 0)


@functools.partial(
    pl.kernel,
    out_type=(jax.ShapeDtypeStruct((NW * EPAD, H), jnp.float32),
              jax.ShapeDtypeStruct((NW * EPAD, H), jnp.float32)),
    mesh=_mesh,
    scratch_types=[
        pltpu.VMEM((ECH,), jnp.int32),
        pltpu.VMEM((ECH, H), jnp.float32),
    ],
)
def _egather_sc(Z, e0, e1, f0, f1, ebuf, fbuf):
    c = lax.axis_index("c")
    s = lax.axis_index("s")
    w = c * NS + s
    for eh, fh in ((e0, f0), (e1, f1)):
        def chunk(k, carry):
            pltpu.sync_copy(eh.at[w, k], ebuf)
            pltpu.sync_copy(Z.at[ebuf], fbuf)
            pltpu.sync_copy(fbuf, fh.at[pl.dslice(w * EPAD + k * ECH, ECH)])
            return carry
        lax.fori_loop(0, ENCH, chunk, 0)


def _mm_body(x_ref, w_ref, o_ref):
    o_ref[...] = jnp.dot(x_ref[...], w_ref[...],
                         preferred_element_type=jnp.float32)


def _tc_matmul(x, w):
    n, dk = x.shape
    dout = w.shape[1]
    bn = 2000
    return pl.pallas_call(
        _mm_body,
        grid=(n // bn,),
        in_specs=[pl.BlockSpec((bn, dk), lambda i: (i, 0)),
                  pl.BlockSpec((dk, dout), lambda i: (0, 0))],
        out_specs=pl.BlockSpec((bn, dout), lambda i: (i, 0)),
        out_shape=jax.ShapeDtypeStruct((n, dout), jnp.float32),
    )(x, w)


def _post_body(add_prev, p_ref, m_ref, wx_ref, prev_ref, g_ref, b_ref, o_ref):
    t = jnp.dot(p_ref[...], wx_ref[...][:H], preferred_element_type=jnp.float32)
    t = t + jnp.dot(m_ref[...], wx_ref[...][H:], preferred_element_type=jnp.float32)
    if add_prev:
        t = t + prev_ref[...]
    mu = jnp.mean(t, axis=0)
    var = jnp.mean((t - mu) ** 2, axis=0)
    h = g_ref[...] * (t - mu) / jnp.sqrt(var + 1e-5) + b_ref[...]
    o_ref[...] = jnp.tanh(h)


def _tc_post(P, M, Wx, prev, g, b, add_prev):
    return pl.pallas_call(
        functools.partial(_post_body, add_prev),
        out_shape=jax.ShapeDtypeStruct((N, H), jnp.float32),
    )(P, M, Wx, prev, g.reshape(1, H), b.reshape(1, H))


def _loss_body(f0_ref, f1_ref, w_ref, y_ref, msk_ref, bd_ref, o_ref):
    i = pl.program_id(0)
    lg = jnp.dot(f0_ref[...], w_ref[...][:, 0],
                 preferred_element_type=jnp.float32)
    lg = lg + jnp.dot(f1_ref[...], w_ref[...][:, 1],
                      preferred_element_type=jnp.float32)
    lg = lg.reshape(y_ref.shape) + bd_ref[0]
    y = y_ref[...]
    t = jnp.maximum(lg, 0.0) - lg * y + jnp.log1p(jnp.exp(-jnp.abs(lg)))
    part = jnp.sum(t * msk_ref[...]) / ES

    @pl.when(i == 0)
    def _():
        o_ref[0, 0] = 0.0
    o_ref[0, 0] = o_ref[0, 0] + part


def _tc_loss(f0, f1, w2, ypad, mask, bdec):
    nb = 8
    rb = f0.shape[0] // nb
    yb = ypad.shape[0] // nb
    out = pl.pallas_call(
        _loss_body,
        grid=(nb,),
        out_shape=jax.ShapeDtypeStruct((1, 1), jnp.float32),
        in_specs=[pl.BlockSpec((rb, H), lambda i: (i, 0)),
                  pl.BlockSpec((rb, H), lambda i: (i, 0)),
                  pl.BlockSpec((H, 2), lambda i: (0, 0)),
                  pl.BlockSpec((yb, H), lambda i: (i, 0)),
                  pl.BlockSpec((yb, H), lambda i: (i, 0)),
                  pl.BlockSpec(memory_space=pltpu.SMEM)],
        out_specs=pl.BlockSpec((1, 1), lambda i: (0, 0),
                               memory_space=pltpu.SMEM),
    )(f0, f1, w2, ypad, mask, bdec)
    return out[0, 0]


def _prep_edges(idx, vals):
    cols = idx[1].reshape(NS, NPT)
    rows = idx[0].reshape(NS, NPT)
    v = vals.reshape(NS, NPT)
    pad = ((0, 0), (0, NCHA * CH - NPT))
    cols = jnp.pad(cols, pad).reshape(NS, NCHA, CH)
    col2 = jnp.stack([cols, cols + NP], axis=0)
    rows = jnp.pad(rows, pad).reshape(NS, NCHA, CH)
    v = jnp.pad(v, pad).reshape(NS, NCHA, CH)
    valx = jnp.broadcast_to(v[..., None], (NS, NCHA, CH, 16))
    return col2, rows, jnp.asarray(valx)


def _pack_state(pred, m0v):
    halves = []
    tx = []
    z = jnp.zeros((N, 64), jnp.float32)
    rpad = ((0, NP - N), (0, 0))
    for c in range(NC):
        sl = slice(64 * c, 64 * (c + 1))
        halves.append(jnp.pad(
            jnp.concatenate([pred[:, sl], m0v[:, sl]], axis=1), rpad))
        tx.append(jnp.pad(
            jnp.concatenate([C * pred[:, sl], z], axis=1), rpad))
    return (jnp.concatenate(halves, axis=0), jnp.concatenate(tx, axis=0))


def _unpack_state(S):
    P = jnp.concatenate([S[0:N, 0:64], S[NP:NP + N, 0:64]], axis=1)
    M = jnp.concatenate([S[0:N, 64:], S[NP:NP + N, 64:]], axis=1)
    return P, M


def kernel(ap_idx, ap_vals, am_idx, am_vals, X, m0, edges, y,
           Ws0, Wx0, bn0_g, bn0_b, Ws1, Wx1, bn1_g, bn1_b, Wdec, bdec):
    colA, rowA, valxA = _prep_edges(ap_idx, ap_vals)
    colB, rowB, valxB = _prep_edges(am_idx, am_vals)

    Ws = [Ws0, Ws1]
    Wx = [Wx0, Wx1]
    gs = [bn0_g, bn1_g]
    bs = [bn0_b, bn1_b]
    prev = X
    cur = X
    for i in range(2):
        pred = _tc_matmul(cur, Ws[i])
        S0, tX = _pack_state(pred, m0)
        S = _diffuse_sc(S0, tX, colA, rowA, valxA, colB, rowB, valxB)
        P, M = _unpack_state(S)
        h = _tc_post(P, M, Wx[i], prev, gs[i], bs[i], i > 0)
        prev = h
        cur = h
    Z = cur

    epad = ((0, 0), (0, EPAD - EPT))
    e0 = jnp.pad(edges[0].reshape(NW, EPT), epad).reshape(NW, ENCH, ECH)
    e1 = jnp.pad(edges[1].reshape(NW, EPT), epad).reshape(NW, ENCH, ECH)
    f0, f1 = _egather_sc(Z, e0, e1)

    ypad = jnp.pad(y.reshape(NW, EPT), epad).reshape(NW * EPAD // H, H)
    mask = jnp.pad(jnp.ones((NW, EPT), jnp.float32), epad)
    mask = mask.reshape(NW * EPAD // H, H)
    w2 = jnp.concatenate([Wdec[:H], Wdec[H:]], axis=1)
    return _tc_loss(f0, f1, w2, ypad, mask, bdec)
